# baseline (speedup 1.0000x reference)
import functools

import jax
import jax.numpy as jnp
from jax import lax
from jax.experimental import pallas as pl
from jax.experimental.pallas import tpu as pltpu
from jax.experimental.pallas import tpu_sc as plsc

N = 10000
D = 128
E = 320000

NC = 2
NS = 16
NW = NC * NS

NPAD = 10240
ROWS_PER_TILE = NPAD // NS

EW = E // NW
EB = 128
NT = 80
EWP = NT * EB
DEGB = 400
DEGT = EW // DEGB

_mesh = lambda: plsc.VectorSubcoreMesh(core_axis_name="c", subcore_axis_name="s")


def _deg_body(dst_hbm, out_hbm, idx_v, ones_v, zb_v, acc, sem):
    c = lax.axis_index("c")
    s = lax.axis_index("s")
    wid = s * NC + c

    for k in range(DEGB // 16):
        ones_v[pl.ds(16 * k, 16)] = jnp.ones((16,), jnp.float32)
    for k in range(ROWS_PER_TILE // 16):
        zb_v[pl.ds(16 * k, 16)] = jnp.zeros((16,), jnp.float32)

    pltpu.sync_copy(zb_v, acc.at[pl.ds(s * ROWS_PER_TILE, ROWS_PER_TILE)])
    plsc.subcore_barrier()

    ebase = wid * EW

    def step(t, carry):
        off = ebase + t * DEGB
        pltpu.sync_copy(dst_hbm.at[pl.ds(off, DEGB)], idx_v)
        pltpu.sync_copy(ones_v, acc.at[idx_v], add=True)
        return carry

    lax.fori_loop(0, DEGT, step, 0)
    plsc.subcore_barrier()

    r0 = s * ROWS_PER_TILE
    pltpu.sync_copy(acc.at[pl.ds(r0, ROWS_PER_TILE)],
                    out_hbm.at[c, pl.ds(r0, ROWS_PER_TILE)])


def _deg_partials(dst):
    kfn = pl.kernel(
        _deg_body,
        out_type=jax.ShapeDtypeStruct((NC, NPAD), jnp.float32),
        mesh=_mesh(),
        scratch_types=[
            pltpu.VMEM((DEGB,), jnp.int32),
            pltpu.VMEM((DEGB,), jnp.float32),
            pltpu.VMEM((ROWS_PER_TILE,), jnp.float32),
            pltpu.VMEM_SHARED((NPAD,), jnp.float32),
            pltpu.SemaphoreType.DMA,
        ],
    )
    return kfn(dst)


def _agg_body(g_hbm, src_hbm, dst_hbm, out_hbm, si_r, di_r, b0, b1, zb_v,
              acc, sem0, sem1, is0, is1, is2, is3, semz):
    c = lax.axis_index("c")
    s = lax.axis_index("s")
    wid = s * NC + c
    isem = (is0, is1, is2, is3)

    for r in range(64):
        for k in range(D // 16):
            zb_v[r, pl.ds(16 * k, 16)] = jnp.zeros((16,), jnp.float32)

    def idx_load(t, slot):
        pltpu.async_copy(src_hbm.at[wid, t], si_r.at[slot], isem[slot])
        pltpu.async_copy(dst_hbm.at[wid, t], di_r.at[slot], isem[slot])

    def idx_wait(slot):
        pltpu.make_async_copy(src_hbm.at[wid, 0], si_r.at[slot],
                              isem[slot]).wait()
        pltpu.make_async_copy(dst_hbm.at[wid, 0], di_r.at[slot],
                              isem[slot]).wait()

    idx_load(0, 0)
    idx_load(1, 1)
    idx_load(2, 2)

    def zstep(t, carry):
        pltpu.async_copy(zb_v, acc.at[pl.ds(s * ROWS_PER_TILE + 64 * t, 64)],
                         semz)
        return carry

    def zwait(t, carry):
        pltpu.make_async_copy(zb_v, acc.at[pl.ds(s * ROWS_PER_TILE, 64)],
                              semz).wait()
        return carry

    lax.fori_loop(0, ROWS_PER_TILE // 64, zstep, 0)
    lax.fori_loop(0, ROWS_PER_TILE // 64, zwait, 0)
    plsc.subcore_barrier()

    def gather(slot, buf, sem):
        pltpu.async_copy(g_hbm.at[si_r.at[slot]], buf, sem)

    def wait(buf, sem):
        pltpu.make_async_copy(g_hbm.at[si_r.at[0]], buf, sem).wait()

    def scatter(slot, buf):
        pltpu.sync_copy(buf, acc.at[di_r.at[slot]], add=True)

    idx_wait(0)
    gather(0, b0, sem0)

    def step(q, carry):
        t0 = 4 * q
        idx_wait(1)
        gather(1, b1, sem1)
        wait(b0, sem0)
        scatter(0, b0)
        idx_load(t0 + 3, 3)
        idx_wait(2)
        gather(2, b0, sem0)
        idx_load(t0 + 4, 0)
        wait(b1, sem1)
        scatter(1, b1)
        idx_wait(3)
        gather(3, b1, sem1)
        wait(b0, sem0)
        scatter(2, b0)
        idx_load(t0 + 5, 1)
        idx_wait(0)
        gather(0, b0, sem0)
        idx_load(t0 + 6, 2)
        wait(b1, sem1)
        scatter(3, b1)
        return carry

    nq = (NT - 4) // 4
    lax.fori_loop(0, nq, step, 0)
    idx_load(NT - 1, 3)
    idx_wait(1)
    gather(1, b1, sem1)
    wait(b0, sem0)
    scatter(0, b0)
    idx_wait(2)
    gather(2, b0, sem0)
    wait(b1, sem1)
    scatter(1, b1)
    idx_wait(3)
    gather(3, b1, sem1)
    wait(b0, sem0)
    scatter(2, b0)
    wait(b1, sem1)
    scatter(3, b1)
    plsc.subcore_barrier()

    r0 = s * ROWS_PER_TILE
    pltpu.sync_copy(acc.at[pl.ds(r0, ROWS_PER_TILE)],
                    out_hbm.at[c, pl.ds(r0, ROWS_PER_TILE)])


def _aggregate(g, src3, dst3):
    assert NT % 4 == 0
    kfn = pl.kernel(
        _agg_body,
        out_type=jax.ShapeDtypeStruct((NC, NPAD, D), jnp.float32),
        mesh=_mesh(),
        scratch_types=[
            pltpu.VMEM((4, EB), jnp.int32),
            pltpu.VMEM((4, EB), jnp.int32),
            pltpu.VMEM((EB, D), jnp.float32),
            pltpu.VMEM((EB, D), jnp.float32),
            pltpu.VMEM((64, D), jnp.float32),
            pltpu.VMEM_SHARED((NPAD, D), jnp.float32),
            pltpu.SemaphoreType.DMA,
            pltpu.SemaphoreType.DMA,
            pltpu.SemaphoreType.DMA,
            pltpu.SemaphoreType.DMA,
            pltpu.SemaphoreType.DMA,
            pltpu.SemaphoreType.DMA,
            pltpu.SemaphoreType.DMA,
        ],
    )
    return kfn(g, src3, dst3)


R = 1000


def _dinv(dega_ref, degb_ref):
    return lax.rsqrt(dega_ref[...] + degb_ref[...] + 1.0)


def _mm_scale_body(x_ref, w_ref, dega_ref, degb_ref, o_ref):
    h = jnp.dot(x_ref[...], w_ref[...], preferred_element_type=jnp.float32)
    o_ref[...] = h * _dinv(dega_ref, degb_ref)


def _combine_mm_body(sa_ref, sb_ref, g_ref, b_ref, w_ref, dega_ref, degb_ref,
                     o_ref):
    dinv = _dinv(dega_ref, degb_ref)
    y = jnp.maximum(dinv * (sa_ref[...] + sb_ref[...] + g_ref[...])
                    + b_ref[...], 0.0)
    o_ref[...] = jnp.dot(y, w_ref[...],
                         preferred_element_type=jnp.float32) * dinv


def _final_body(sa_ref, sb_ref, g_ref, b_ref, dega_ref, degb_ref, o_ref):
    dinv = _dinv(dega_ref, degb_ref)
    o_ref[...] = dinv * (sa_ref[...] + sb_ref[...] + g_ref[...]) + b_ref[...]


_row_spec = pl.BlockSpec((R, D), lambda i: (i, 0))
_col_spec = pl.BlockSpec((R, 1), lambda i: (i, 0))
_w_spec = pl.BlockSpec((D, D), lambda i: (0, 0))
_b_spec = pl.BlockSpec((1, D), lambda i: (0, 0))
_out_shape = jax.ShapeDtypeStruct((N, D), jnp.float32)


def _mm_scale(x, w, dega, degb):
    return pl.pallas_call(
        _mm_scale_body,
        grid=(N // R,),
        in_specs=[_row_spec, _w_spec, _col_spec, _col_spec],
        out_specs=_row_spec,
        out_shape=_out_shape,
    )(x, w, dega, degb)


def _combine_mm(sa, sb, g, b, w, dega, degb):
    return pl.pallas_call(
        _combine_mm_body,
        grid=(N // R,),
        in_specs=[_row_spec, _row_spec, _row_spec, _b_spec, _w_spec,
                  _col_spec, _col_spec],
        out_specs=_row_spec,
        out_shape=_out_shape,
    )(sa, sb, g, b, w, dega, degb)


def _final(sa, sb, g, b, dega, degb):
    return pl.pallas_call(
        _final_body,
        grid=(N // R,),
        in_specs=[_row_spec, _row_spec, _row_spec, _b_spec,
                  _col_spec, _col_spec],
        out_specs=_row_spec,
        out_shape=_out_shape,
    )(sa, sb, g, b, dega, degb)


def kernel(x, edge_index, W1, b1, W2, b2, W3, b3, W4, b4):
    src = edge_index[0]
    dst = edge_index[1]
    padw = EWP - EW
    k = jnp.arange(padw, dtype=jnp.int32)
    pad_src = jnp.broadcast_to((k * 89) % N, (NW, padw))
    pad_dst = jnp.broadcast_to(N + (k % (NPAD - N)), (NW, padw))
    src3 = jnp.concatenate([src.reshape(NW, EW), pad_src], 1).reshape(NW, NT, EB)
    dst3 = jnp.concatenate([dst.reshape(NW, EW), pad_dst], 1).reshape(NW, NT, EB)

    deg_p = _deg_partials(dst)
    dega = deg_p[0, :N, None]
    degb = deg_p[1, :N, None]

    g = _mm_scale(x, W1, dega, degb)
    for (b_l, w_next) in ((b1, W2), (b2, W3), (b3, W4)):
        s_p = _aggregate(g, src3, dst3)
        g = _combine_mm(s_p[0, :N], s_p[1, :N], g, b_l.reshape(1, D), w_next,
                        dega, degb)
    s_p = _aggregate(g, src3, dst3)
    return _final(s_p[0, :N], s_p[1, :N], g, b4.reshape(1, D), dega, degb)


# speedup vs baseline: 1.0561x; 1.0561x over previous
import functools

import jax
import jax.numpy as jnp
from jax import lax
from jax.experimental import pallas as pl
from jax.experimental.pallas import tpu as pltpu
from jax.experimental.pallas import tpu_sc as plsc

N = 10000
D = 128
E = 320000

NC = 2
NS = 16
NW = NC * NS

NPAD = 10240
ROWS_PER_TILE = NPAD // NS

EW = E // NW
EB = 128
NT = 80
EWP = NT * EB
DEGB = 400
DEGT = EW // DEGB

_mesh = lambda: plsc.VectorSubcoreMesh(core_axis_name="c", subcore_axis_name="s")


def _deg_body(dst_hbm, out_hbm, idx_v, ones_v, zb_v, acc, sem):
    c = lax.axis_index("c")
    s = lax.axis_index("s")
    wid = s * NC + c

    for k in range(DEGB // 16):
        ones_v[pl.ds(16 * k, 16)] = jnp.ones((16,), jnp.float32)
    for k in range(ROWS_PER_TILE // 16):
        zb_v[pl.ds(16 * k, 16)] = jnp.zeros((16,), jnp.float32)

    pltpu.sync_copy(zb_v, acc.at[pl.ds(s * ROWS_PER_TILE, ROWS_PER_TILE)])
    plsc.subcore_barrier()

    ebase = wid * EW

    def step(t, carry):
        off = ebase + t * DEGB
        pltpu.sync_copy(dst_hbm.at[pl.ds(off, DEGB)], idx_v)
        pltpu.sync_copy(ones_v, acc.at[idx_v], add=True)
        return carry

    lax.fori_loop(0, DEGT, step, 0)
    plsc.subcore_barrier()

    r0 = s * ROWS_PER_TILE
    pltpu.sync_copy(acc.at[pl.ds(r0, ROWS_PER_TILE)],
                    out_hbm.at[c, pl.ds(r0, ROWS_PER_TILE)])


def _deg_partials(dst):
    kfn = pl.kernel(
        _deg_body,
        out_type=jax.ShapeDtypeStruct((NC, NPAD), jnp.float32),
        mesh=_mesh(),
        scratch_types=[
            pltpu.VMEM((DEGB,), jnp.int32),
            pltpu.VMEM((DEGB,), jnp.float32),
            pltpu.VMEM((ROWS_PER_TILE,), jnp.float32),
            pltpu.VMEM_SHARED((NPAD,), jnp.float32),
            pltpu.SemaphoreType.DMA,
        ],
    )
    return kfn(dst)


def _agg_body(g_hbm, src_hbm, dst_hbm, out_hbm, si_r, di_r, b0, b1, zb_v,
              acc, sem0, sem1, is0, is1, is2, is3, semz):
    c = lax.axis_index("c")
    s = lax.axis_index("s")
    wid = s * NC + c
    isem = (is0, is1, is2, is3)

    for r in range(64):
        for k in range(D // 16):
            zb_v[r, pl.ds(16 * k, 16)] = jnp.zeros((16,), jnp.float32)

    def idx_load(t, slot):
        pltpu.async_copy(src_hbm.at[wid, t], si_r.at[slot], isem[slot])
        pltpu.async_copy(dst_hbm.at[wid, t], di_r.at[slot], isem[slot])

    def idx_wait(slot):
        pltpu.make_async_copy(src_hbm.at[wid, 0], si_r.at[slot],
                              isem[slot]).wait()
        pltpu.make_async_copy(dst_hbm.at[wid, 0], di_r.at[slot],
                              isem[slot]).wait()

    idx_load(0, 0)
    idx_load(1, 1)
    idx_load(2, 2)

    def zstep(t, carry):
        pltpu.async_copy(zb_v, acc.at[pl.ds(s * ROWS_PER_TILE + 64 * t, 64)],
                         semz)
        return carry

    def zwait(t, carry):
        pltpu.make_async_copy(zb_v, acc.at[pl.ds(s * ROWS_PER_TILE, 64)],
                              semz).wait()
        return carry

    lax.fori_loop(0, ROWS_PER_TILE // 64, zstep, 0)
    lax.fori_loop(0, ROWS_PER_TILE // 64, zwait, 0)
    plsc.subcore_barrier()

    def gather(slot, buf, sem):
        pltpu.async_copy(g_hbm.at[si_r.at[slot]], buf, sem)

    def wait(buf, sem):
        pltpu.make_async_copy(g_hbm.at[si_r.at[0]], buf, sem).wait()

    def scatter(slot, buf):
        pltpu.sync_copy(buf, acc.at[di_r.at[slot]], add=True)

    idx_wait(0)
    gather(0, b0, sem0)

    def step(q, carry):
        t0 = 4 * q
        idx_wait(1)
        gather(1, b1, sem1)
        wait(b0, sem0)
        scatter(0, b0)
        idx_load(t0 + 3, 3)
        idx_wait(2)
        gather(2, b0, sem0)
        idx_load(t0 + 4, 0)
        wait(b1, sem1)
        scatter(1, b1)
        idx_wait(3)
        gather(3, b1, sem1)
        wait(b0, sem0)
        scatter(2, b0)
        idx_load(t0 + 5, 1)
        idx_wait(0)
        gather(0, b0, sem0)
        idx_load(t0 + 6, 2)
        wait(b1, sem1)
        scatter(3, b1)
        return carry

    nq = (NT - 4) // 4
    lax.fori_loop(0, nq, step, 0)
    idx_load(NT - 1, 3)
    idx_wait(1)
    gather(1, b1, sem1)
    wait(b0, sem0)
    scatter(0, b0)
    idx_wait(2)
    gather(2, b0, sem0)
    wait(b1, sem1)
    scatter(1, b1)
    idx_wait(3)
    gather(3, b1, sem1)
    wait(b0, sem0)
    scatter(2, b0)
    wait(b1, sem1)
    scatter(3, b1)
    plsc.subcore_barrier()

    r0 = s * ROWS_PER_TILE
    pltpu.sync_copy(acc.at[pl.ds(r0, ROWS_PER_TILE)],
                    out_hbm.at[c, pl.ds(r0, ROWS_PER_TILE)])


def _aggregate(g, src3, dst3):
    assert NT % 4 == 0
    kfn = pl.kernel(
        _agg_body,
        out_type=jax.ShapeDtypeStruct((NC, NPAD, D), jnp.float32),
        mesh=_mesh(),
        scratch_types=[
            pltpu.VMEM((4, EB), jnp.int32),
            pltpu.VMEM((4, EB), jnp.int32),
            pltpu.VMEM((EB, D), jnp.float32),
            pltpu.VMEM((EB, D), jnp.float32),
            pltpu.VMEM((64, D), jnp.float32),
            pltpu.VMEM_SHARED((NPAD, D), jnp.float32),
            pltpu.SemaphoreType.DMA,
            pltpu.SemaphoreType.DMA,
            pltpu.SemaphoreType.DMA,
            pltpu.SemaphoreType.DMA,
            pltpu.SemaphoreType.DMA,
            pltpu.SemaphoreType.DMA,
            pltpu.SemaphoreType.DMA,
        ],
    )
    return kfn(g, src3, dst3)


R = 1000


def _dinv(dega_ref, degb_ref):
    return lax.rsqrt(dega_ref[...] + degb_ref[...] + 1.0)


def _mm_scale_body(x_ref, w_ref, dega_ref, degb_ref, o_ref):
    h = jnp.dot(x_ref[...], w_ref[...], preferred_element_type=jnp.float32)
    o_ref[...] = h * _dinv(dega_ref, degb_ref)


def _combine_mm_body(sp_ref, g_ref, b_ref, w_ref, dega_ref, degb_ref,
                     o_ref):
    dinv = _dinv(dega_ref, degb_ref)
    y = jnp.maximum(dinv * (sp_ref[0] + sp_ref[1] + g_ref[...])
                    + b_ref[...], 0.0)
    o_ref[...] = jnp.dot(y, w_ref[...],
                         preferred_element_type=jnp.float32) * dinv


def _final_body(sp_ref, g_ref, b_ref, dega_ref, degb_ref, o_ref):
    dinv = _dinv(dega_ref, degb_ref)
    o_ref[...] = dinv * (sp_ref[0] + sp_ref[1] + g_ref[...]) + b_ref[...]


_row_spec = pl.BlockSpec((R, D), lambda i: (i, 0))
_sp_spec = pl.BlockSpec((NC, R, D), lambda i: (0, i, 0))
_col_spec = pl.BlockSpec((R, 1), lambda i: (i, 0))
_w_spec = pl.BlockSpec((D, D), lambda i: (0, 0))
_b_spec = pl.BlockSpec((1, D), lambda i: (0, 0))
_out_shape = jax.ShapeDtypeStruct((N, D), jnp.float32)


def _mm_scale(x, w, dega, degb):
    return pl.pallas_call(
        _mm_scale_body,
        grid=(N // R,),
        in_specs=[_row_spec, _w_spec, _col_spec, _col_spec],
        out_specs=_row_spec,
        out_shape=_out_shape,
    )(x, w, dega, degb)


def _combine_mm(sp, g, b, w, dega, degb):
    return pl.pallas_call(
        _combine_mm_body,
        grid=(N // R,),
        in_specs=[_sp_spec, _row_spec, _b_spec, _w_spec,
                  _col_spec, _col_spec],
        out_specs=_row_spec,
        out_shape=_out_shape,
    )(sp, g, b, w, dega, degb)


def _final(sp, g, b, dega, degb):
    return pl.pallas_call(
        _final_body,
        grid=(N // R,),
        in_specs=[_sp_spec, _row_spec, _b_spec,
                  _col_spec, _col_spec],
        out_specs=_row_spec,
        out_shape=_out_shape,
    )(sp, g, b, dega, degb)


def kernel(x, edge_index, W1, b1, W2, b2, W3, b3, W4, b4):
    src = edge_index[0]
    dst = edge_index[1]
    padw = EWP - EW
    k = jnp.arange(padw, dtype=jnp.int32)
    pad_src = jnp.broadcast_to((k * 89) % N, (NW, padw))
    pad_dst = jnp.broadcast_to(N + (k % (NPAD - N)), (NW, padw))
    src3 = jnp.concatenate([src.reshape(NW, EW), pad_src], 1).reshape(NW, NT, EB)
    dst3 = jnp.concatenate([dst.reshape(NW, EW), pad_dst], 1).reshape(NW, NT, EB)

    deg_p = _deg_partials(dst)
    dega = deg_p[0, :N, None]
    degb = deg_p[1, :N, None]

    g = _mm_scale(x, W1, dega, degb)
    for (b_l, w_next) in ((b1, W2), (b2, W3), (b3, W4)):
        s_p = _aggregate(g, src3, dst3)
        g = _combine_mm(s_p, g, b_l.reshape(1, D), w_next, dega, degb)
    s_p = _aggregate(g, src3, dst3)
    return _final(s_p, g, b4.reshape(1, D), dega, degb)


# speedup vs baseline: 1.0682x; 1.0115x over previous
import functools

import jax
import jax.numpy as jnp
from jax import lax
from jax.experimental import pallas as pl
from jax.experimental.pallas import tpu as pltpu
from jax.experimental.pallas import tpu_sc as plsc

N = 10000
D = 128
E = 320000

NC = 2
NS = 16
NW = NC * NS

NPAD = 10240
ROWS_PER_TILE = NPAD // NS

EW = E // NW
EB = 128
NT = 79
EWP = NT * EB
DEGB = 400
DEGT = EW // DEGB

_mesh = lambda: plsc.VectorSubcoreMesh(core_axis_name="c", subcore_axis_name="s")


def _deg_body(dst_hbm, out_hbm, idx_v, ones_v, zb_v, acc, sem):
    c = lax.axis_index("c")
    s = lax.axis_index("s")
    wid = s * NC + c

    for k in range(DEGB // 16):
        ones_v[pl.ds(16 * k, 16)] = jnp.ones((16,), jnp.float32)
    for k in range(ROWS_PER_TILE // 16):
        zb_v[pl.ds(16 * k, 16)] = jnp.zeros((16,), jnp.float32)

    pltpu.sync_copy(zb_v, acc.at[pl.ds(s * ROWS_PER_TILE, ROWS_PER_TILE)])
    plsc.subcore_barrier()

    ebase = wid * EW

    def step(t, carry):
        off = ebase + t * DEGB
        pltpu.sync_copy(dst_hbm.at[pl.ds(off, DEGB)], idx_v)
        pltpu.sync_copy(ones_v, acc.at[idx_v], add=True)
        return carry

    lax.fori_loop(0, DEGT, step, 0)
    plsc.subcore_barrier()

    r0 = s * ROWS_PER_TILE
    pltpu.sync_copy(acc.at[pl.ds(r0, ROWS_PER_TILE)],
                    out_hbm.at[c, pl.ds(r0, ROWS_PER_TILE)])


def _deg_partials(dst):
    kfn = pl.kernel(
        _deg_body,
        out_type=jax.ShapeDtypeStruct((NC, NPAD), jnp.float32),
        mesh=_mesh(),
        scratch_types=[
            pltpu.VMEM((DEGB,), jnp.int32),
            pltpu.VMEM((DEGB,), jnp.float32),
            pltpu.VMEM((ROWS_PER_TILE,), jnp.float32),
            pltpu.VMEM_SHARED((NPAD,), jnp.float32),
            pltpu.SemaphoreType.DMA,
        ],
    )
    return kfn(dst)


def _agg_body(g_hbm, src_hbm, dst_hbm, out_hbm, si_r, di_r, b0, b1, zb_v,
              acc, sem0, sem1, is0, is1, is2, is3, semz):
    c = lax.axis_index("c")
    s = lax.axis_index("s")
    wid = s * NC + c
    isem = (is0, is1, is2, is3)

    for r in range(64):
        for k in range(D // 16):
            zb_v[r, pl.ds(16 * k, 16)] = jnp.zeros((16,), jnp.float32)

    def idx_load(t, slot):
        pltpu.async_copy(src_hbm.at[wid, t], si_r.at[slot], isem[slot])
        pltpu.async_copy(dst_hbm.at[wid, t], di_r.at[slot], isem[slot])

    def idx_wait(slot):
        pltpu.make_async_copy(src_hbm.at[wid, 0], si_r.at[slot],
                              isem[slot]).wait()
        pltpu.make_async_copy(dst_hbm.at[wid, 0], di_r.at[slot],
                              isem[slot]).wait()

    idx_load(0, 0)
    idx_load(1, 1)
    idx_load(2, 2)

    def zstep(t, carry):
        pltpu.async_copy(zb_v, acc.at[pl.ds(s * ROWS_PER_TILE + 64 * t, 64)],
                         semz)
        return carry

    def zwait(t, carry):
        pltpu.make_async_copy(zb_v, acc.at[pl.ds(s * ROWS_PER_TILE, 64)],
                              semz).wait()
        return carry

    lax.fori_loop(0, ROWS_PER_TILE // 64, zstep, 0)
    lax.fori_loop(0, ROWS_PER_TILE // 64, zwait, 0)
    plsc.subcore_barrier()

    def gather(slot, buf, sem):
        pltpu.async_copy(g_hbm.at[si_r.at[slot]], buf, sem)

    def wait(buf, sem):
        pltpu.make_async_copy(g_hbm.at[si_r.at[0]], buf, sem).wait()

    def scatter(slot, buf):
        pltpu.sync_copy(buf, acc.at[di_r.at[slot]], add=True)

    idx_wait(0)
    gather(0, b0, sem0)

    def step(q, carry):
        t0 = 4 * q
        idx_wait(1)
        gather(1, b1, sem1)
        wait(b0, sem0)
        scatter(0, b0)
        idx_load(t0 + 3, 3)
        idx_wait(2)
        gather(2, b0, sem0)
        idx_load(t0 + 4, 0)
        wait(b1, sem1)
        scatter(1, b1)
        idx_wait(3)
        gather(3, b1, sem1)
        wait(b0, sem0)
        scatter(2, b0)
        idx_load(t0 + 5, 1)
        idx_wait(0)
        gather(0, b0, sem0)
        idx_load(t0 + 6, 2)
        wait(b1, sem1)
        scatter(3, b1)
        return carry

    nq = (NT - 3) // 4
    lax.fori_loop(0, nq, step, 0)
    idx_wait(1)
    gather(1, b1, sem1)
    wait(b0, sem0)
    scatter(0, b0)
    idx_wait(2)
    gather(2, b0, sem0)
    wait(b1, sem1)
    scatter(1, b1)
    wait(b0, sem0)
    scatter(2, b0)
    plsc.subcore_barrier()

    r0 = s * ROWS_PER_TILE
    pltpu.sync_copy(acc.at[pl.ds(r0, ROWS_PER_TILE)],
                    out_hbm.at[c, pl.ds(r0, ROWS_PER_TILE)])


def _aggregate(g, src3, dst3):
    assert NT % 4 == 3
    kfn = pl.kernel(
        _agg_body,
        out_type=jax.ShapeDtypeStruct((NC, NPAD, D), jnp.float32),
        mesh=_mesh(),
        scratch_types=[
            pltpu.VMEM((4, EB), jnp.int32),
            pltpu.VMEM((4, EB), jnp.int32),
            pltpu.VMEM((EB, D), jnp.float32),
            pltpu.VMEM((EB, D), jnp.float32),
            pltpu.VMEM((64, D), jnp.float32),
            pltpu.VMEM_SHARED((NPAD, D), jnp.float32),
            pltpu.SemaphoreType.DMA,
            pltpu.SemaphoreType.DMA,
            pltpu.SemaphoreType.DMA,
            pltpu.SemaphoreType.DMA,
            pltpu.SemaphoreType.DMA,
            pltpu.SemaphoreType.DMA,
            pltpu.SemaphoreType.DMA,
        ],
    )
    return kfn(g, src3, dst3)


R = 1000


def _dinv(dega_ref, degb_ref):
    return lax.rsqrt(dega_ref[...] + degb_ref[...] + 1.0)


def _mm_scale_body(x_ref, w_ref, dega_ref, degb_ref, o_ref):
    h = jnp.dot(x_ref[...], w_ref[...], preferred_element_type=jnp.float32)
    o_ref[...] = h * _dinv(dega_ref, degb_ref)


def _combine_mm_body(sp_ref, g_ref, b_ref, w_ref, dega_ref, degb_ref,
                     o_ref):
    dinv = _dinv(dega_ref, degb_ref)
    y = jnp.maximum(dinv * (sp_ref[0] + sp_ref[1] + g_ref[...])
                    + b_ref[...], 0.0)
    o_ref[...] = jnp.dot(y, w_ref[...],
                         preferred_element_type=jnp.float32) * dinv


def _final_body(sp_ref, g_ref, b_ref, dega_ref, degb_ref, o_ref):
    dinv = _dinv(dega_ref, degb_ref)
    o_ref[...] = dinv * (sp_ref[0] + sp_ref[1] + g_ref[...]) + b_ref[...]


_row_spec = pl.BlockSpec((R, D), lambda i: (i, 0))
_sp_spec = pl.BlockSpec((NC, R, D), lambda i: (0, i, 0))
_col_spec = pl.BlockSpec((R, 1), lambda i: (i, 0))
_w_spec = pl.BlockSpec((D, D), lambda i: (0, 0))
_b_spec = pl.BlockSpec((1, D), lambda i: (0, 0))
_out_shape = jax.ShapeDtypeStruct((N, D), jnp.float32)


def _mm_scale(x, w, dega, degb):
    return pl.pallas_call(
        _mm_scale_body,
        grid=(N // R,),
        in_specs=[_row_spec, _w_spec, _col_spec, _col_spec],
        out_specs=_row_spec,
        out_shape=_out_shape,
    )(x, w, dega, degb)


def _combine_mm(sp, g, b, w, dega, degb):
    return pl.pallas_call(
        _combine_mm_body,
        grid=(N // R,),
        in_specs=[_sp_spec, _row_spec, _b_spec, _w_spec,
                  _col_spec, _col_spec],
        out_specs=_row_spec,
        out_shape=_out_shape,
    )(sp, g, b, w, dega, degb)


def _final(sp, g, b, dega, degb):
    return pl.pallas_call(
        _final_body,
        grid=(N // R,),
        in_specs=[_sp_spec, _row_spec, _b_spec,
                  _col_spec, _col_spec],
        out_specs=_row_spec,
        out_shape=_out_shape,
    )(sp, g, b, dega, degb)


def kernel(x, edge_index, W1, b1, W2, b2, W3, b3, W4, b4):
    src = edge_index[0]
    dst = edge_index[1]
    padw = EWP - EW
    k = jnp.arange(padw, dtype=jnp.int32)
    pad_src = jnp.broadcast_to((k * 89) % N, (NW, padw))
    pad_dst = jnp.broadcast_to(N + (k % (NPAD - N)), (NW, padw))
    src3 = jnp.concatenate([src.reshape(NW, EW), pad_src], 1).reshape(NW, NT, EB)
    dst3 = jnp.concatenate([dst.reshape(NW, EW), pad_dst], 1).reshape(NW, NT, EB)

    deg_p = _deg_partials(dst)
    dega = deg_p[0, :N, None]
    degb = deg_p[1, :N, None]

    g = _mm_scale(x, W1, dega, degb)
    for (b_l, w_next) in ((b1, W2), (b2, W3), (b3, W4)):
        s_p = _aggregate(g, src3, dst3)
        g = _combine_mm(s_p, g, b_l.reshape(1, D), w_next, dega, degb)
    s_p = _aggregate(g, src3, dst3)
    return _final(s_p, g, b4.reshape(1, D), dega, degb)


# speedup vs baseline: 1.0815x; 1.0125x over previous
import functools

import jax
import jax.numpy as jnp
from jax import lax
from jax.experimental import pallas as pl
from jax.experimental.pallas import tpu as pltpu
from jax.experimental.pallas import tpu_sc as plsc

N = 10000
D = 128
E = 320000

NC = 2
NS = 16
NW = NC * NS

NPAD = 10240
ROWS_PER_TILE = NPAD // NS

EW = E // NW
EB = 128
NT = 79
EWP = NT * EB
DEGB = 400
DEGT = EW // DEGB

_mesh = lambda: plsc.VectorSubcoreMesh(core_axis_name="c", subcore_axis_name="s")


def _deg_body(dst_hbm, out_hbm, idx_v, ones_v, zb_v, acc, sem):
    c = lax.axis_index("c")
    s = lax.axis_index("s")
    wid = s * NC + c

    for k in range(DEGB // 16):
        ones_v[pl.ds(16 * k, 16)] = jnp.ones((16,), jnp.float32)
    for k in range(ROWS_PER_TILE // 16):
        zb_v[pl.ds(16 * k, 16)] = jnp.zeros((16,), jnp.float32)

    pltpu.sync_copy(zb_v, acc.at[pl.ds(s * ROWS_PER_TILE, ROWS_PER_TILE)])
    plsc.subcore_barrier()

    ebase = wid * EW

    def step(t, carry):
        off = ebase + t * DEGB
        pltpu.sync_copy(dst_hbm.at[pl.ds(off, DEGB)], idx_v)
        pltpu.sync_copy(ones_v, acc.at[idx_v], add=True)
        return carry

    lax.fori_loop(0, DEGT, step, 0)
    plsc.subcore_barrier()

    r0 = s * ROWS_PER_TILE
    pltpu.sync_copy(acc.at[pl.ds(r0, ROWS_PER_TILE)],
                    out_hbm.at[c, pl.ds(r0, ROWS_PER_TILE)])


def _deg_partials(dst):
    kfn = pl.kernel(
        _deg_body,
        out_type=jax.ShapeDtypeStruct((NC, NPAD), jnp.float32),
        mesh=_mesh(),
        scratch_types=[
            pltpu.VMEM((DEGB,), jnp.int32),
            pltpu.VMEM((DEGB,), jnp.float32),
            pltpu.VMEM((ROWS_PER_TILE,), jnp.float32),
            pltpu.VMEM_SHARED((NPAD,), jnp.float32),
            pltpu.SemaphoreType.DMA,
        ],
    )
    return kfn(dst)


def _agg_body(g_hbm, src_hbm, dst_hbm, out_hbm, si_r, di_r, b0, b1, zb_v,
              acc, sem0, sem1, is0, is1, is2, is3, semz):
    c = lax.axis_index("c")
    s = lax.axis_index("s")
    wid = s * NC + c
    isem = (is0, is1, is2, is3)

    for r in range(64):
        for k in range(D // 16):
            zb_v[r, pl.ds(16 * k, 16)] = jnp.zeros((16,), jnp.float32)

    def idx_load(t, slot):
        pltpu.async_copy(src_hbm.at[wid, t], si_r.at[slot], isem[slot])
        pltpu.async_copy(dst_hbm.at[wid, t], di_r.at[slot], isem[slot])

    def idx_wait(slot):
        pltpu.make_async_copy(src_hbm.at[wid, 0], si_r.at[slot],
                              isem[slot]).wait()
        pltpu.make_async_copy(dst_hbm.at[wid, 0], di_r.at[slot],
                              isem[slot]).wait()

    idx_load(0, 0)
    idx_load(1, 1)
    idx_load(2, 2)

    def zstep(t, carry):
        pltpu.async_copy(zb_v, acc.at[pl.ds(s * ROWS_PER_TILE + 64 * t, 64)],
                         semz)
        return carry

    def zwait(t, carry):
        pltpu.make_async_copy(zb_v, acc.at[pl.ds(s * ROWS_PER_TILE, 64)],
                              semz).wait()
        return carry

    def gather(slot, buf, sem):
        pltpu.async_copy(g_hbm.at[si_r.at[slot]], buf, sem)

    def wait(buf, sem):
        pltpu.make_async_copy(g_hbm.at[si_r.at[0]], buf, sem).wait()

    def scatter(slot, buf):
        pltpu.sync_copy(buf, acc.at[di_r.at[slot]], add=True)

    lax.fori_loop(0, ROWS_PER_TILE // 64, zstep, 0)
    idx_wait(0)
    gather(0, b0, sem0)
    lax.fori_loop(0, ROWS_PER_TILE // 64, zwait, 0)
    plsc.subcore_barrier()

    def step(q, carry):
        t0 = 4 * q
        idx_wait(1)
        gather(1, b1, sem1)
        wait(b0, sem0)
        scatter(0, b0)
        idx_load(t0 + 3, 3)
        idx_wait(2)
        gather(2, b0, sem0)
        idx_load(t0 + 4, 0)
        wait(b1, sem1)
        scatter(1, b1)
        idx_wait(3)
        gather(3, b1, sem1)
        wait(b0, sem0)
        scatter(2, b0)
        idx_load(t0 + 5, 1)
        idx_wait(0)
        gather(0, b0, sem0)
        idx_load(t0 + 6, 2)
        wait(b1, sem1)
        scatter(3, b1)
        return carry

    nq = (NT - 3) // 4
    lax.fori_loop(0, nq, step, 0)
    idx_wait(1)
    gather(1, b1, sem1)
    wait(b0, sem0)
    scatter(0, b0)
    idx_wait(2)
    gather(2, b0, sem0)
    wait(b1, sem1)
    scatter(1, b1)
    wait(b0, sem0)
    scatter(2, b0)
    plsc.subcore_barrier()

    r0 = s * ROWS_PER_TILE
    pltpu.sync_copy(acc.at[pl.ds(r0, ROWS_PER_TILE)],
                    out_hbm.at[c, pl.ds(r0, ROWS_PER_TILE)])


def _aggregate(g, src3, dst3):
    assert NT % 4 == 3
    kfn = pl.kernel(
        _agg_body,
        out_type=jax.ShapeDtypeStruct((NC, NPAD, D), jnp.float32),
        mesh=_mesh(),
        scratch_types=[
            pltpu.VMEM((4, EB), jnp.int32),
            pltpu.VMEM((4, EB), jnp.int32),
            pltpu.VMEM((EB, D), jnp.float32),
            pltpu.VMEM((EB, D), jnp.float32),
            pltpu.VMEM((64, D), jnp.float32),
            pltpu.VMEM_SHARED((NPAD, D), jnp.float32),
            pltpu.SemaphoreType.DMA,
            pltpu.SemaphoreType.DMA,
            pltpu.SemaphoreType.DMA,
            pltpu.SemaphoreType.DMA,
            pltpu.SemaphoreType.DMA,
            pltpu.SemaphoreType.DMA,
            pltpu.SemaphoreType.DMA,
        ],
    )
    return kfn(g, src3, dst3)


R = 1000


def _dinv(dega_ref, degb_ref):
    return lax.rsqrt(dega_ref[...] + degb_ref[...] + 1.0)


def _mm_scale_body(x_ref, w_ref, dega_ref, degb_ref, o_ref):
    h = jnp.dot(x_ref[...], w_ref[...], preferred_element_type=jnp.float32)
    o_ref[...] = h * _dinv(dega_ref, degb_ref)


def _combine_mm_body(sp_ref, g_ref, b_ref, w_ref, dega_ref, degb_ref,
                     o_ref):
    dinv = _dinv(dega_ref, degb_ref)
    y = jnp.maximum(dinv * (sp_ref[0] + sp_ref[1] + g_ref[...])
                    + b_ref[...], 0.0)
    o_ref[...] = jnp.dot(y, w_ref[...],
                         preferred_element_type=jnp.float32) * dinv


def _final_body(sp_ref, g_ref, b_ref, dega_ref, degb_ref, o_ref):
    dinv = _dinv(dega_ref, degb_ref)
    o_ref[...] = dinv * (sp_ref[0] + sp_ref[1] + g_ref[...]) + b_ref[...]


_row_spec = pl.BlockSpec((R, D), lambda i: (i, 0))
_sp_spec = pl.BlockSpec((NC, R, D), lambda i: (0, i, 0))
_col_spec = pl.BlockSpec((R, 1), lambda i: (i, 0))
_w_spec = pl.BlockSpec((D, D), lambda i: (0, 0))
_b_spec = pl.BlockSpec((1, D), lambda i: (0, 0))
_out_shape = jax.ShapeDtypeStruct((N, D), jnp.float32)


def _mm_scale(x, w, dega, degb):
    return pl.pallas_call(
        _mm_scale_body,
        grid=(N // R,),
        in_specs=[_row_spec, _w_spec, _col_spec, _col_spec],
        out_specs=_row_spec,
        out_shape=_out_shape,
    )(x, w, dega, degb)


def _combine_mm(sp, g, b, w, dega, degb):
    return pl.pallas_call(
        _combine_mm_body,
        grid=(N // R,),
        in_specs=[_sp_spec, _row_spec, _b_spec, _w_spec,
                  _col_spec, _col_spec],
        out_specs=_row_spec,
        out_shape=_out_shape,
    )(sp, g, b, w, dega, degb)


def _final(sp, g, b, dega, degb):
    return pl.pallas_call(
        _final_body,
        grid=(N // R,),
        in_specs=[_sp_spec, _row_spec, _b_spec,
                  _col_spec, _col_spec],
        out_specs=_row_spec,
        out_shape=_out_shape,
    )(sp, g, b, dega, degb)


def kernel(x, edge_index, W1, b1, W2, b2, W3, b3, W4, b4):
    src = edge_index[0]
    dst = edge_index[1]
    padw = EWP - EW
    k = jnp.arange(padw, dtype=jnp.int32)
    pad_src = jnp.broadcast_to((k * 89) % N, (NW, padw))
    pad_dst = jnp.broadcast_to(N + (k % (NPAD - N)), (NW, padw))
    src3 = jnp.concatenate([src.reshape(NW, EW), pad_src], 1).reshape(NW, NT, EB)
    dst3 = jnp.concatenate([dst.reshape(NW, EW), pad_dst], 1).reshape(NW, NT, EB)

    deg_p = _deg_partials(dst)
    dega = deg_p[0, :N, None]
    degb = deg_p[1, :N, None]

    g = _mm_scale(x, W1, dega, degb)
    for (b_l, w_next) in ((b1, W2), (b2, W3), (b3, W4)):
        s_p = _aggregate(g, src3, dst3)
        g = _combine_mm(s_p, g, b_l.reshape(1, D), w_next, dega, degb)
    s_p = _aggregate(g, src3, dst3)
    return _final(s_p, g, b4.reshape(1, D), dega, degb)


# speedup vs baseline: 1.0953x; 1.0128x over previous
import functools

import jax
import jax.numpy as jnp
from jax import lax
from jax.experimental import pallas as pl
from jax.experimental.pallas import tpu as pltpu
from jax.experimental.pallas import tpu_sc as plsc

N = 10000
D = 128
E = 320000

NC = 2
NS = 16
NW = NC * NS

NPAD = 10240
ROWS_PER_TILE = NPAD // NS

EW = E // NW
EB = 128
NT = 79
EWP = NT * EB
_mesh = lambda: plsc.VectorSubcoreMesh(core_axis_name="c", subcore_axis_name="s")


def _deg_body(dst_hbm, out_hbm, di_v, ones_v, zb_v, acc, sem):
    c = lax.axis_index("c")
    s = lax.axis_index("s")
    wid = s * NC + c

    for k in range(EB // 16):
        ones_v[pl.ds(16 * k, 16)] = jnp.ones((16,), jnp.float32)
    for k in range(ROWS_PER_TILE // 16):
        zb_v[pl.ds(16 * k, 16)] = jnp.zeros((16,), jnp.float32)

    pltpu.async_copy(dst_hbm.at[wid], di_v, sem)
    pltpu.sync_copy(zb_v, acc.at[pl.ds(s * ROWS_PER_TILE, ROWS_PER_TILE)])
    pltpu.make_async_copy(dst_hbm.at[wid], di_v, sem).wait()
    plsc.subcore_barrier()

    def step(t, carry):
        pltpu.sync_copy(ones_v, acc.at[di_v.at[t]], add=True)
        return carry

    lax.fori_loop(0, NT, step, 0)
    plsc.subcore_barrier()

    r0 = s * ROWS_PER_TILE
    pltpu.sync_copy(acc.at[pl.ds(r0, ROWS_PER_TILE)],
                    out_hbm.at[c, pl.ds(r0, ROWS_PER_TILE)])


def _deg_partials(dst3):
    kfn = pl.kernel(
        _deg_body,
        out_type=jax.ShapeDtypeStruct((NC, NPAD), jnp.float32),
        mesh=_mesh(),
        scratch_types=[
            pltpu.VMEM((NT, EB), jnp.int32),
            pltpu.VMEM((EB,), jnp.float32),
            pltpu.VMEM((ROWS_PER_TILE,), jnp.float32),
            pltpu.VMEM_SHARED((NPAD,), jnp.float32),
            pltpu.SemaphoreType.DMA,
        ],
    )
    return kfn(dst3)


def _agg_body(g_hbm, src_hbm, dst_hbm, out_hbm, si_r, di_r, b0, b1, zb_v,
              acc, sem0, sem1, is0, is1, is2, is3, semz):
    c = lax.axis_index("c")
    s = lax.axis_index("s")
    wid = s * NC + c
    isem = (is0, is1, is2, is3)

    for r in range(64):
        for k in range(D // 16):
            zb_v[r, pl.ds(16 * k, 16)] = jnp.zeros((16,), jnp.float32)

    def idx_load(t, slot):
        pltpu.async_copy(src_hbm.at[wid, t], si_r.at[slot], isem[slot])
        pltpu.async_copy(dst_hbm.at[wid, t], di_r.at[slot], isem[slot])

    def idx_wait(slot):
        pltpu.make_async_copy(src_hbm.at[wid, 0], si_r.at[slot],
                              isem[slot]).wait()
        pltpu.make_async_copy(dst_hbm.at[wid, 0], di_r.at[slot],
                              isem[slot]).wait()

    idx_load(0, 0)
    idx_load(1, 1)
    idx_load(2, 2)

    def zstep(t, carry):
        pltpu.async_copy(zb_v, acc.at[pl.ds(s * ROWS_PER_TILE + 64 * t, 64)],
                         semz)
        return carry

    def zwait(t, carry):
        pltpu.make_async_copy(zb_v, acc.at[pl.ds(s * ROWS_PER_TILE, 64)],
                              semz).wait()
        return carry

    def gather(slot, buf, sem):
        pltpu.async_copy(g_hbm.at[si_r.at[slot]], buf, sem)

    def wait(buf, sem):
        pltpu.make_async_copy(g_hbm.at[si_r.at[0]], buf, sem).wait()

    def scatter(slot, buf):
        pltpu.sync_copy(buf, acc.at[di_r.at[slot]], add=True)

    lax.fori_loop(0, ROWS_PER_TILE // 64, zstep, 0)
    idx_wait(0)
    gather(0, b0, sem0)
    lax.fori_loop(0, ROWS_PER_TILE // 64, zwait, 0)
    plsc.subcore_barrier()

    def step(q, carry):
        t0 = 4 * q
        idx_wait(1)
        gather(1, b1, sem1)
        wait(b0, sem0)
        scatter(0, b0)
        idx_load(t0 + 3, 3)
        idx_wait(2)
        gather(2, b0, sem0)
        idx_load(t0 + 4, 0)
        wait(b1, sem1)
        scatter(1, b1)
        idx_wait(3)
        gather(3, b1, sem1)
        wait(b0, sem0)
        scatter(2, b0)
        idx_load(t0 + 5, 1)
        idx_wait(0)
        gather(0, b0, sem0)
        idx_load(t0 + 6, 2)
        wait(b1, sem1)
        scatter(3, b1)
        return carry

    nq = (NT - 3) // 4
    lax.fori_loop(0, nq, step, 0)
    idx_wait(1)
    gather(1, b1, sem1)
    wait(b0, sem0)
    scatter(0, b0)
    idx_wait(2)
    gather(2, b0, sem0)
    wait(b1, sem1)
    scatter(1, b1)
    wait(b0, sem0)
    scatter(2, b0)
    plsc.subcore_barrier()

    r0 = s * ROWS_PER_TILE
    pltpu.sync_copy(acc.at[pl.ds(r0, ROWS_PER_TILE)],
                    out_hbm.at[c, pl.ds(r0, ROWS_PER_TILE)])


def _aggregate(g, src3, dst3):
    assert NT % 4 == 3
    kfn = pl.kernel(
        _agg_body,
        out_type=jax.ShapeDtypeStruct((NC, NPAD, D), jnp.float32),
        mesh=_mesh(),
        scratch_types=[
            pltpu.VMEM((4, EB), jnp.int32),
            pltpu.VMEM((4, EB), jnp.int32),
            pltpu.VMEM((EB, D), jnp.float32),
            pltpu.VMEM((EB, D), jnp.float32),
            pltpu.VMEM((64, D), jnp.float32),
            pltpu.VMEM_SHARED((NPAD, D), jnp.float32),
            pltpu.SemaphoreType.DMA,
            pltpu.SemaphoreType.DMA,
            pltpu.SemaphoreType.DMA,
            pltpu.SemaphoreType.DMA,
            pltpu.SemaphoreType.DMA,
            pltpu.SemaphoreType.DMA,
            pltpu.SemaphoreType.DMA,
        ],
    )
    return kfn(g, src3, dst3)


R = 1000


def _dinv(dega_ref, degb_ref):
    return lax.rsqrt(dega_ref[...] + degb_ref[...] + 1.0)


def _mm_scale_body(x_ref, w_ref, dega_ref, degb_ref, o_ref):
    h = jnp.dot(x_ref[...], w_ref[...], preferred_element_type=jnp.float32)
    o_ref[...] = h * _dinv(dega_ref, degb_ref)


def _combine_mm_body(sp_ref, g_ref, b_ref, w_ref, dega_ref, degb_ref,
                     o_ref):
    dinv = _dinv(dega_ref, degb_ref)
    y = jnp.maximum(dinv * (sp_ref[0] + sp_ref[1] + g_ref[...])
                    + b_ref[...], 0.0)
    o_ref[...] = jnp.dot(y, w_ref[...],
                         preferred_element_type=jnp.float32) * dinv


def _final_body(sp_ref, g_ref, b_ref, dega_ref, degb_ref, o_ref):
    dinv = _dinv(dega_ref, degb_ref)
    o_ref[...] = dinv * (sp_ref[0] + sp_ref[1] + g_ref[...]) + b_ref[...]


_row_spec = pl.BlockSpec((R, D), lambda i: (i, 0))
_sp_spec = pl.BlockSpec((NC, R, D), lambda i: (0, i, 0))
_col_spec = pl.BlockSpec((R, 1), lambda i: (i, 0))
_w_spec = pl.BlockSpec((D, D), lambda i: (0, 0))
_b_spec = pl.BlockSpec((1, D), lambda i: (0, 0))
_out_shape = jax.ShapeDtypeStruct((N, D), jnp.float32)


def _mm_scale(x, w, dega, degb):
    return pl.pallas_call(
        _mm_scale_body,
        grid=(N // R,),
        in_specs=[_row_spec, _w_spec, _col_spec, _col_spec],
        out_specs=_row_spec,
        out_shape=_out_shape,
    )(x, w, dega, degb)


def _combine_mm(sp, g, b, w, dega, degb):
    return pl.pallas_call(
        _combine_mm_body,
        grid=(N // R,),
        in_specs=[_sp_spec, _row_spec, _b_spec, _w_spec,
                  _col_spec, _col_spec],
        out_specs=_row_spec,
        out_shape=_out_shape,
    )(sp, g, b, w, dega, degb)


def _final(sp, g, b, dega, degb):
    return pl.pallas_call(
        _final_body,
        grid=(N // R,),
        in_specs=[_sp_spec, _row_spec, _b_spec,
                  _col_spec, _col_spec],
        out_specs=_row_spec,
        out_shape=_out_shape,
    )(sp, g, b, dega, degb)


def kernel(x, edge_index, W1, b1, W2, b2, W3, b3, W4, b4):
    src = edge_index[0]
    dst = edge_index[1]
    padw = EWP - EW
    k = jnp.arange(padw, dtype=jnp.int32)
    pad_src = jnp.broadcast_to((k * 89) % N, (NW, padw))
    pad_dst = jnp.broadcast_to(N + (k % (NPAD - N)), (NW, padw))
    src3 = jnp.concatenate([src.reshape(NW, EW), pad_src], 1).reshape(NW, NT, EB)
    dst3 = jnp.concatenate([dst.reshape(NW, EW), pad_dst], 1).reshape(NW, NT, EB)

    deg_p = _deg_partials(dst3)
    dega = deg_p[0, :N, None]
    degb = deg_p[1, :N, None]

    g = _mm_scale(x, W1, dega, degb)
    for (b_l, w_next) in ((b1, W2), (b2, W3), (b3, W4)):
        s_p = _aggregate(g, src3, dst3)
        g = _combine_mm(s_p, g, b_l.reshape(1, D), w_next, dega, degb)
    s_p = _aggregate(g, src3, dst3)
    return _final(s_p, g, b4.reshape(1, D), dega, degb)
